# baseline (speedup 1.0000x reference)
import jax
import jax.numpy as jnp
from jax import lax
from jax.experimental import pallas as pl
from jax.experimental.pallas import tpu as pltpu
from jax.experimental.pallas import tpu_sc as plsc

VOCAB = 1000000
DIM = 64
BATCH = 16384

NUM_CORES = 2
NUM_SUBCORES = 16
NUM_WORKERS = NUM_CORES * NUM_SUBCORES
TD_PER_CORE = 4
BPS = BATCH // NUM_SUBCORES
BPW = BATCH // NUM_WORKERS
IDX_CHUNK = 128
NCHUNK = BPS // IDX_CHUNK
LANES = 16

_ALIGNED = (VOCAB // 128) * 128
_NTILES = _ALIGNED // 128


def _mk_pieces(n):
    base, rem = divmod(_NTILES, n)
    pieces, off = [], 0
    for i in range(n):
        ln = (base + (1 if i < rem else 0)) * 128
        pieces.append((off, ln))
        off += ln
    return pieces


STAGE_PIECES = _mk_pieces(8)
TAIL_OFF = _ALIGNED
TAIL_LEN = VOCAB - _ALIGNED
TAIL_PAD = 128
SP_LEN = _ALIGNED + TAIL_PAD


def _dot_kernel_body(idx0_hbm, idx1_hbm, t0_hbm, t1_hbm, tail0_hbm,
                     tail1_hbm, parts_hbm, sp0, sp1, idx0_v, idx1_v,
                     v0_v, v1_v, acc_v, sem):
    cid = lax.axis_index("c")
    sid = lax.axis_index("s")

    pltpu.sync_copy(idx0_hbm.at[pl.ds(sid * NCHUNK, NCHUNK)], idx0_v)
    pltpu.sync_copy(idx1_hbm.at[pl.ds(sid * NCHUNK, NCHUNK)], idx1_v)


    def zero_step(i, carry):
        acc_v[pl.ds(i * LANES, LANES)] = jnp.zeros((LANES,), jnp.float32)
        return carry

    lax.fori_loop(0, BPS // LANES, zero_step, 0)

    def td_step(tdl, carry):
        td = cid * TD_PER_CORE + tdl
        for sd in range(8):
            for p, (off, ln) in enumerate(STAGE_PIECES):
                @pl.when(sid == p)
                def _(off=off, ln=ln):
                    pltpu.sync_copy(t0_hbm.at[td, sd, pl.ds(off, ln)],
                                    sp0.at[pl.ds(off, ln)])

                @pl.when(sid == 8 + p)
                def _(off=off, ln=ln):
                    pltpu.sync_copy(t1_hbm.at[td, sd, pl.ds(off, ln)],
                                    sp1.at[pl.ds(off, ln)])

            dglob = td * 8 + sd

            @pl.when(sid == 0)
            def _():
                pltpu.sync_copy(
                    tail0_hbm.at[pl.ds(dglob * TAIL_PAD, TAIL_PAD)],
                    sp0.at[pl.ds(TAIL_OFF, TAIL_PAD)])

            @pl.when(sid == 8)
            def _():
                pltpu.sync_copy(
                    tail1_hbm.at[pl.ds(dglob * TAIL_PAD, TAIL_PAD)],
                    sp1.at[pl.ds(TAIL_OFF, TAIL_PAD)])

            plsc.subcore_barrier()

            copies = []
            for c in range(NCHUNK):
                dst = pl.ds(c * IDX_CHUNK, IDX_CHUNK)
                copies.append(pltpu.async_copy(
                    sp0.at[idx0_v.at[c]], v0_v.at[dst], sem))
                copies.append(pltpu.async_copy(
                    sp1.at[idx1_v.at[c]], v1_v.at[dst], sem))
            for cp in copies:
                cp.wait()

            plsc.subcore_barrier()

            def acc_step(i, c2):
                s = pl.ds(i * LANES, LANES)
                acc_v[s] = acc_v[s] + v0_v[s] * v1_v[s]
                return c2

            lax.fori_loop(0, BPS // LANES, acc_step, 0)
        return carry

    lax.fori_loop(0, TD_PER_CORE, td_step, 0)

    pltpu.sync_copy(
        acc_v, parts_hbm.at[pl.ds(cid * BATCH + sid * BPS, BPS)])


def _loss_kernel_body(parts_hbm, labels_hbm, out_hbm, p0_v, p1_v, lab_v,
                      part_v):
    wid = lax.axis_index("s") * NUM_CORES + lax.axis_index("c")
    base = wid * BPW

    pltpu.sync_copy(parts_hbm.at[pl.ds(base, BPW)], p0_v)
    pltpu.sync_copy(parts_hbm.at[pl.ds(BATCH + base, BPW)], p1_v)
    pltpu.sync_copy(labels_hbm.at[pl.ds(base, BPW)], lab_v)

    def loss_step(g, loss_acc):
        s = pl.ds(g * LANES, LANES)
        pred = p0_v[s] + p1_v[s]
        sig = 1.0 / (1.0 + jnp.exp(-pred))
        diff = sig - lab_v[s]
        return loss_acc + diff * diff

    loss_acc = lax.fori_loop(0, BPW // LANES, loss_step,
                             jnp.zeros((LANES,), jnp.float32))

    part_v[...] = loss_acc
    pltpu.sync_copy(part_v, out_hbm.at[pl.ds(wid * LANES, LANES)])


@jax.jit
def _run(idx0, idx1, labels, t0, t1, tail0, tail1):
    mesh = plsc.VectorSubcoreMesh(core_axis_name="c", subcore_axis_name="s")
    parts = pl.kernel(
        _dot_kernel_body,
        out_type=jax.ShapeDtypeStruct((NUM_CORES * BATCH,), jnp.float32),
        mesh=mesh,
        compiler_params=pltpu.CompilerParams(
            needs_layout_passes=False, use_tc_tiling_on_sc=True),
        scratch_types=[
            pltpu.VMEM_SHARED((SP_LEN,), jnp.float32),
            pltpu.VMEM_SHARED((SP_LEN,), jnp.float32),
            pltpu.VMEM((NCHUNK, IDX_CHUNK), jnp.int32),
            pltpu.VMEM((NCHUNK, IDX_CHUNK), jnp.int32),
            pltpu.VMEM((BPS,), jnp.float32),
            pltpu.VMEM((BPS,), jnp.float32),
            pltpu.VMEM((BPS,), jnp.float32),
            pltpu.SemaphoreType.DMA,
        ],
    )(idx0, idx1, t0, t1, tail0, tail1)

    losses = pl.kernel(
        _loss_kernel_body,
        out_type=jax.ShapeDtypeStruct((NUM_WORKERS * LANES,), jnp.float32),
        mesh=mesh,
        compiler_params=pltpu.CompilerParams(
            needs_layout_passes=False, use_tc_tiling_on_sc=True),
        scratch_types=[
            pltpu.VMEM((BPW,), jnp.float32),
            pltpu.VMEM((BPW,), jnp.float32),
            pltpu.VMEM((BPW,), jnp.float32),
            pltpu.VMEM((LANES,), jnp.float32),
        ],
    )(parts, labels)

    return jnp.sum(losses) * (1.0 / BATCH)


def kernel(indices_f0, indices_f1, labels, emb_table_0, emb_table_1):
    idx0 = indices_f0.astype(jnp.int32).reshape(BATCH // IDX_CHUNK,
                                                IDX_CHUNK)
    idx1 = indices_f1.astype(jnp.int32).reshape(BATCH // IDX_CHUNK,
                                                IDX_CHUNK)
    e0t = emb_table_0.T
    e1t = emb_table_1.T
    t0 = e0t.reshape(8, 8, VOCAB)
    t1 = e1t.reshape(8, 8, VOCAB)
    tail0 = jnp.pad(e0t[:, TAIL_OFF:],
                    ((0, 0), (0, TAIL_PAD - TAIL_LEN))).reshape(-1)
    tail1 = jnp.pad(e1t[:, TAIL_OFF:],
                    ((0, 0), (0, TAIL_PAD - TAIL_LEN))).reshape(-1)
    return _run(idx0, idx1, labels, t0, t1, tail0, tail1)


# speedup vs baseline: 1.0776x; 1.0776x over previous
import jax
import jax.numpy as jnp
from jax import lax
from jax.experimental import pallas as pl
from jax.experimental.pallas import tpu as pltpu
from jax.experimental.pallas import tpu_sc as plsc

VOCAB = 1000000
DIM = 64
BATCH = 16384

NUM_CORES = 2
NUM_SUBCORES = 16
NUM_WORKERS = NUM_CORES * NUM_SUBCORES
TD_PER_CORE = 4
BPS = BATCH // NUM_SUBCORES
BPW = BATCH // NUM_WORKERS
IDX_CHUNK = 128
NCHUNK = BPS // IDX_CHUNK
LANES = 16

_ALIGNED = (VOCAB // 128) * 128
_NTILES = _ALIGNED // 128


def _mk_pieces(n):
    base, rem = divmod(_NTILES, n)
    pieces, off = [], 0
    for i in range(n):
        ln = (base + (1 if i < rem else 0)) * 128
        pieces.append((off, ln))
        off += ln
    return pieces


STAGE_PIECES = _mk_pieces(8)
TAIL_OFF = _ALIGNED
TAIL_LEN = VOCAB - _ALIGNED
TAIL_PAD = 128
SP_LEN = _ALIGNED + TAIL_PAD


def _dot_kernel_body(idx0_hbm, idx1_hbm, t0_hbm, t1_hbm, tail0_hbm,
                     tail1_hbm, parts_hbm, sp0, sp1, idx0_v, idx1_v,
                     v0_v, v1_v, acc_v, sem, sem2):
    cid = lax.axis_index("c")
    sid = lax.axis_index("s")

    pltpu.sync_copy(idx0_hbm.at[pl.ds(sid * NCHUNK, NCHUNK)], idx0_v)
    pltpu.sync_copy(idx1_hbm.at[pl.ds(sid * NCHUNK, NCHUNK)], idx1_v)

    def zero_step(i, carry):
        acc_v[pl.ds(i * LANES, LANES)] = jnp.zeros((LANES,), jnp.float32)
        return carry

    lax.fori_loop(0, BPS // LANES, zero_step, 0)

    def stage_async(t_hbm, tail_hbm, sp, td, sd):
        for p, (off, ln) in enumerate(STAGE_PIECES):
            @pl.when(sid == p)
            def _(off=off, ln=ln):
                pltpu.async_copy(t_hbm.at[td, sd, pl.ds(off, ln)],
                                 sp.at[pl.ds(off, ln)], sem2)

        @pl.when(sid == 0)
        def _():
            pltpu.async_copy(
                tail_hbm.at[pl.ds((td * 8 + sd) * TAIL_PAD, TAIL_PAD)],
                sp.at[pl.ds(TAIL_OFF, TAIL_PAD)], sem2)

    def stage_drain(t_hbm, tail_hbm, sp, td, sd):
        for p, (off, ln) in enumerate(STAGE_PIECES):
            @pl.when(sid == p)
            def _(off=off, ln=ln):
                pltpu.make_async_copy(t_hbm.at[td, sd, pl.ds(off, ln)],
                                      sp.at[pl.ds(off, ln)], sem2).wait()

        @pl.when(sid == 0)
        def _():
            pltpu.make_async_copy(
                tail_hbm.at[pl.ds((td * 8 + sd) * TAIL_PAD, TAIL_PAD)],
                sp.at[pl.ds(TAIL_OFF, TAIL_PAD)], sem2).wait()

    def gather(sp, idx_v, v_v):
        copies = []
        for c in range(NCHUNK):
            dst = pl.ds(c * IDX_CHUNK, IDX_CHUNK)
            copies.append(pltpu.async_copy(
                sp.at[idx_v.at[c]], v_v.at[dst], sem))
        for cp in copies:
            cp.wait()

    td0 = cid * TD_PER_CORE
    stage_async(t0_hbm, tail0_hbm, sp0, td0, 0)
    stage_drain(t0_hbm, tail0_hbm, sp0, td0, 0)
    plsc.subcore_barrier()

    def td_step(tdl, carry):
        td = cid * TD_PER_CORE + tdl
        for sd in range(8):
            stage_async(t1_hbm, tail1_hbm, sp1, td, sd)
            gather(sp0, idx0_v, v0_v)
            stage_drain(t1_hbm, tail1_hbm, sp1, td, sd)
            plsc.subcore_barrier()

            if sd < 7:
                stage_async(t0_hbm, tail0_hbm, sp0, td, sd + 1)
                gather(sp1, idx1_v, v1_v)
                stage_drain(t0_hbm, tail0_hbm, sp0, td, sd + 1)
            else:
                @pl.when(tdl < TD_PER_CORE - 1)
                def _():
                    stage_async(t0_hbm, tail0_hbm, sp0, td + 1, 0)
                gather(sp1, idx1_v, v1_v)

                @pl.when(tdl < TD_PER_CORE - 1)
                def _():
                    stage_drain(t0_hbm, tail0_hbm, sp0, td + 1, 0)
            plsc.subcore_barrier()

            def acc_step(i, c2):
                s = pl.ds(i * LANES, LANES)
                acc_v[s] = acc_v[s] + v0_v[s] * v1_v[s]
                return c2

            lax.fori_loop(0, BPS // LANES, acc_step, 0)
        return carry

    lax.fori_loop(0, TD_PER_CORE, td_step, 0)

    pltpu.sync_copy(
        acc_v, parts_hbm.at[pl.ds(cid * BATCH + sid * BPS, BPS)])


def _loss_kernel_body(parts_hbm, labels_hbm, out_hbm, p0_v, p1_v, lab_v,
                      part_v):
    wid = lax.axis_index("s") * NUM_CORES + lax.axis_index("c")
    base = wid * BPW

    pltpu.sync_copy(parts_hbm.at[pl.ds(base, BPW)], p0_v)
    pltpu.sync_copy(parts_hbm.at[pl.ds(BATCH + base, BPW)], p1_v)
    pltpu.sync_copy(labels_hbm.at[pl.ds(base, BPW)], lab_v)

    def loss_step(g, loss_acc):
        s = pl.ds(g * LANES, LANES)
        pred = p0_v[s] + p1_v[s]
        sig = 1.0 / (1.0 + jnp.exp(-pred))
        diff = sig - lab_v[s]
        return loss_acc + diff * diff

    loss_acc = lax.fori_loop(0, BPW // LANES, loss_step,
                             jnp.zeros((LANES,), jnp.float32))

    part_v[...] = loss_acc
    pltpu.sync_copy(part_v, out_hbm.at[pl.ds(wid * LANES, LANES)])


@jax.jit
def _run(idx0, idx1, labels, t0, t1, tail0, tail1):
    mesh = plsc.VectorSubcoreMesh(core_axis_name="c", subcore_axis_name="s")
    parts = pl.kernel(
        _dot_kernel_body,
        out_type=jax.ShapeDtypeStruct((NUM_CORES * BATCH,), jnp.float32),
        mesh=mesh,
        compiler_params=pltpu.CompilerParams(
            needs_layout_passes=False, use_tc_tiling_on_sc=True),
        scratch_types=[
            pltpu.VMEM_SHARED((SP_LEN,), jnp.float32),
            pltpu.VMEM_SHARED((SP_LEN,), jnp.float32),
            pltpu.VMEM((NCHUNK, IDX_CHUNK), jnp.int32),
            pltpu.VMEM((NCHUNK, IDX_CHUNK), jnp.int32),
            pltpu.VMEM((BPS,), jnp.float32),
            pltpu.VMEM((BPS,), jnp.float32),
            pltpu.VMEM((BPS,), jnp.float32),
            pltpu.SemaphoreType.DMA,
            pltpu.SemaphoreType.DMA,
        ],
    )(idx0, idx1, t0, t1, tail0, tail1)

    losses = pl.kernel(
        _loss_kernel_body,
        out_type=jax.ShapeDtypeStruct((NUM_WORKERS * LANES,), jnp.float32),
        mesh=mesh,
        compiler_params=pltpu.CompilerParams(
            needs_layout_passes=False, use_tc_tiling_on_sc=True),
        scratch_types=[
            pltpu.VMEM((BPW,), jnp.float32),
            pltpu.VMEM((BPW,), jnp.float32),
            pltpu.VMEM((BPW,), jnp.float32),
            pltpu.VMEM((LANES,), jnp.float32),
        ],
    )(parts, labels)

    return jnp.sum(losses) * (1.0 / BATCH)


def kernel(indices_f0, indices_f1, labels, emb_table_0, emb_table_1):
    idx0 = indices_f0.astype(jnp.int32).reshape(BATCH // IDX_CHUNK,
                                                IDX_CHUNK)
    idx1 = indices_f1.astype(jnp.int32).reshape(BATCH // IDX_CHUNK,
                                                IDX_CHUNK)
    e0t = emb_table_0.T
    e1t = emb_table_1.T
    t0 = e0t.reshape(8, 8, VOCAB)
    t1 = e1t.reshape(8, 8, VOCAB)
    tail0 = jnp.pad(e0t[:, TAIL_OFF:],
                    ((0, 0), (0, TAIL_PAD - TAIL_LEN))).reshape(-1)
    tail1 = jnp.pad(e1t[:, TAIL_OFF:],
                    ((0, 0), (0, TAIL_PAD - TAIL_LEN))).reshape(-1)
    return _run(idx0, idx1, labels, t0, t1, tail0, tail1)
